# baseline (speedup 1.0000x reference)
import jax
import jax.numpy as jnp
from jax import lax
from jax.experimental import pallas as pl
from jax.experimental.pallas import tpu as pltpu
from jax.experimental.pallas import tpu_sc as plsc

BATCH = 16384
EMBED = 32
LANES = 16
CHUNK = 128
TBL_WORDS = 1000000 * EMBED
DS_LEN = TBL_WORDS - EMBED + 8

_info = plsc.get_sparse_core_info()
_NC = _info.num_cores
_NS = _info.num_subcores
_NW = _NC * _NS
_BPW = BATCH // _NW
_NCHUNK = _BPW // CHUNK


def _sc_body(uid_hbm, iid_hbm, ut_hbm, it_hbm, out_hbm,
             uidx_v, iidx_v, ub_v, ib_v, uT, iT, out_v,
             sem_u0, sem_u1, sem_i0, sem_i1):
    wid = lax.axis_index("s") * _NC + lax.axis_index("c")
    base = wid * _BPW
    sems_u = (sem_u0, sem_u1)
    sems_i = (sem_i0, sem_i1)

    pltpu.sync_copy(uid_hbm.at[pl.ds(base, _BPW)], uidx_v)
    pltpu.sync_copy(iid_hbm.at[pl.ds(base, _BPW)], iidx_v)

    def to_words(j, carry):
        sl = pl.ds(j * LANES, LANES)
        uw = lax.shift_left(uidx_v[sl], 5)
        iw = lax.shift_left(iidx_v[sl], 5)
        for d in range(8):
            ub_v[d, sl] = uw + d
            ib_v[d, sl] = iw + d
        return carry

    lax.fori_loop(0, _BPW // LANES, to_words, 0)

    def start(j):
        slot = j % 2
        copies = []
        for c in range(EMBED):
            hi, d = 8 * (c // 8), c % 8
            isl = (d, pl.ds(j * CHUNK, CHUNK))
            copies.append(pltpu.async_copy(
                ut_hbm.at[pl.ds(hi, DS_LEN)].at[ub_v.at[isl]],
                uT.at[slot, c], sems_u[slot]))
            copies.append(pltpu.async_copy(
                it_hbm.at[pl.ds(hi, DS_LEN)].at[ib_v.at[isl]],
                iT.at[slot, c], sems_i[slot]))
        return copies

    inflight = start(0)
    for j in range(_NCHUNK):
        nxt = start(j + 1) if j + 1 < _NCHUNK else None
        for cp in inflight:
            cp.wait()
        slot = j % 2
        for g in range(CHUNK // LANES):
            gsl = pl.ds(g * LANES, LANES)
            acc = uT[slot, 0, gsl] * iT[slot, 0, gsl]
            for c in range(1, EMBED):
                acc = acc + uT[slot, c, gsl] * iT[slot, c, gsl]
            out_v[pl.ds(j * CHUNK + g * LANES, LANES)] = acc
        inflight = nxt

    pltpu.sync_copy(out_v, out_hbm.at[pl.ds(base, _BPW)])


@jax.jit
def _impl(user_ids, item_ids, user_table, item_table):
    mesh = plsc.VectorSubcoreMesh(core_axis_name="c", subcore_axis_name="s")
    f = pl.kernel(
        _sc_body,
        out_type=jax.ShapeDtypeStruct((BATCH,), jnp.float32),
        mesh=mesh,
        compiler_params=pltpu.CompilerParams(
            needs_layout_passes=False, use_tc_tiling_on_sc=False),
        scratch_types=[
            pltpu.VMEM((_BPW,), jnp.int32),
            pltpu.VMEM((_BPW,), jnp.int32),
            pltpu.VMEM((8, _BPW), jnp.int32),
            pltpu.VMEM((8, _BPW), jnp.int32),
            pltpu.VMEM((2, EMBED, CHUNK), jnp.float32),
            pltpu.VMEM((2, EMBED, CHUNK), jnp.float32),
            pltpu.VMEM((_BPW,), jnp.float32),
            pltpu.SemaphoreType.DMA,
            pltpu.SemaphoreType.DMA,
            pltpu.SemaphoreType.DMA,
            pltpu.SemaphoreType.DMA,
        ],
    )
    ut = user_table.reshape(-1)
    it = item_table.reshape(-1)
    return f(user_ids, item_ids, ut, it)


def kernel(user_ids, item_ids, user_table, item_table):
    return _impl(user_ids.astype(jnp.int32), item_ids.astype(jnp.int32),
                 user_table, item_table)
